# baseline (speedup 1.0000x reference)
import functools

import jax
import jax.numpy as jnp
from jax import lax
from jax.experimental import pallas as pl
from jax.experimental.pallas import tpu as pltpu
from jax.experimental.pallas import tpu_sc as plsc

NC = 2
NS = 16
NW = NC * NS
LANES = 16
CH = 32


def kernel(x, pos_table):
    B, L, D = x.shape
    seq_per_w = L // NW
    nchunk = seq_per_w // CH
    nvec = D // LANES

    mesh = plsc.VectorSubcoreMesh(core_axis_name="c", subcore_axis_name="s")

    @functools.partial(
        pl.kernel,
        out_type=jax.ShapeDtypeStruct((B * L, D), jnp.float32),
        mesh=mesh,
        scratch_types=[
            pltpu.VMEM((CH, D), jnp.float32),
            pltpu.VMEM((CH, D), jnp.float32),
            pltpu.VMEM((CH, D), jnp.float32),
            pltpu.SemaphoreType.DMA,
            pltpu.SemaphoreType.DMA,
            pltpu.SemaphoreType.DMA,
            pltpu.SemaphoreType.DMA,
        ],
    )
    def sc_add(x_hbm, pos_hbm, out_hbm, pb, xb0, xb1, si0, si1, so0, so1):
        c = lax.axis_index("c")
        s = lax.axis_index("s")
        wid = s * NC + c
        base = wid * seq_per_w
        xb = (xb0, xb1)
        si = (si0, si1)
        so = (so0, so1)

        def in_copy(j, b, p):
            row0 = base + j * CH
            return pltpu.make_async_copy(
                x_hbm.at[pl.ds(b * L + row0, CH)], xb[p], si[p])

        def out_copy(j, b, p):
            row0 = base + j * CH
            return pltpu.make_async_copy(
                xb[p], out_hbm.at[pl.ds(b * L + row0, CH)], so[p])

        in_copy(0, 0, 0).start()

        def chunk_body(j, carry):
            pltpu.sync_copy(pos_hbm.at[pl.ds(base + j * CH, CH)], pb)
            for b in range(B):
                p = b % 2
                q = 1 - p
                if b + 1 < B:
                    @pl.when(jnp.logical_or(j > 0, b > 0))
                    def _():
                        out_copy(j, b - 1, q).wait()
                    in_copy(j, b + 1, q).start()
                else:
                    @pl.when(j < nchunk - 1)
                    def _():
                        out_copy(j, b - 1, q).wait()
                        in_copy(j + 1, 0, q).start()
                in_copy(j, b, p).wait()

                @plsc.parallel_loop(0, CH, 1, unroll=2)
                def row_body(r):
                    for o in range(nvec):
                        sl = pl.ds(o * LANES, LANES)
                        xb[p][r, sl] = xb[p][r, sl] + pb[r, sl]
                out_copy(j, b, p).start()
            return carry

        lax.fori_loop(0, nchunk, chunk_body, 0)
        out_copy(nchunk - 1, B - 2, 0).wait()
        out_copy(nchunk - 1, B - 1, 1).wait()

    out = sc_add(x.reshape(B * L, D), pos_table)
    return out.reshape(B, L, D)


# speedup vs baseline: 2.2186x; 2.2186x over previous
import jax
import jax.numpy as jnp
from jax.experimental import pallas as pl

BLK = 2048


def _add_kernel(x_ref, pos_ref, out_ref):
    out_ref[...] = x_ref[...] + pos_ref[...][None, :, :]


def kernel(x, pos_table):
    B, L, D = x.shape
    grid = (L // BLK, B)
    return pl.pallas_call(
        _add_kernel,
        grid=grid,
        in_specs=[
            pl.BlockSpec((1, BLK, D), lambda i, b: (b, i, 0)),
            pl.BlockSpec((BLK, D), lambda i, b: (i, 0)),
        ],
        out_specs=pl.BlockSpec((1, BLK, D), lambda i, b: (b, i, 0)),
        out_shape=jax.ShapeDtypeStruct((B, L, D), x.dtype),
    )(x, pos_table)


# speedup vs baseline: 2.2193x; 1.0003x over previous
import jax
import jax.numpy as jnp
from jax.experimental import pallas as pl

BLK = 1024


def _add_kernel(x_ref, pos_ref, out_ref):
    out_ref[...] = x_ref[...] + pos_ref[...][None, :, :]


def kernel(x, pos_table):
    B, L, D = x.shape
    grid = (L // BLK, B // 2)
    return pl.pallas_call(
        _add_kernel,
        grid=grid,
        in_specs=[
            pl.BlockSpec((2, BLK, D), lambda i, b: (b, i, 0)),
            pl.BlockSpec((BLK, D), lambda i, b: (i, 0)),
        ],
        out_specs=pl.BlockSpec((2, BLK, D), lambda i, b: (b, i, 0)),
        out_shape=jax.ShapeDtypeStruct((B, L, D), x.dtype),
    )(x, pos_table)
